# baseline (speedup 1.0000x reference)
import functools

import jax
import jax.numpy as jnp
from jax import lax
from jax.experimental import pallas as pl
from jax.experimental.pallas import tpu as pltpu
from jax.experimental.pallas import tpu_sc as plsc

_NC = 2
_NS = 16


def _shuffle(img2, inds2, *, C, HW, npairs):
    cpc = C // _NC
    ppt = npairs // _NS
    hpt = ppt // 2
    slw = HW // _NS

    mesh = plsc.VectorSubcoreMesh(core_axis_name="c", subcore_axis_name="s")

    @functools.partial(
        pl.kernel,
        out_type=jax.ShapeDtypeStruct((C, HW), jnp.float32),
        mesh=mesh,
        scratch_types=[
            pltpu.VMEM((ppt,), jnp.int32),
            pltpu.VMEM((ppt,), jnp.int32),
            pltpu.VMEM((ppt,), jnp.int32),
            pltpu.VMEM((ppt,), jnp.int32),
            pltpu.VMEM((ppt,), jnp.float32),
            pltpu.VMEM((ppt,), jnp.float32),
            pltpu.VMEM_SHARED((HW,), jnp.float32),
            pltpu.VMEM_SHARED((HW,), jnp.float32),
            pltpu.VMEM_SHARED((HW,), jnp.float32),
            pltpu.SemaphoreType.DMA,
            pltpu.SemaphoreType.DMA,
            pltpu.SemaphoreType.DMA,
            pltpu.SemaphoreType.DMA,
            pltpu.SemaphoreType.DMA,
            pltpu.SemaphoreType.DMA,
            pltpu.SemaphoreType.DMA,
            pltpu.SemaphoreType.DMA,
            pltpu.SemaphoreType.DMA,
        ],
    )
    def run(img_hbm, inds_hbm, out_hbm, g0i, g1i, s0i, s1i, v0, v1,
            rowA, rowB, rowC, sinA, sinB, sinC, soutA, soutB, soutC,
            sg0, sg1, ss):
        sc = lax.axis_index("c")
        t = lax.axis_index("s")
        ch0 = sc * cpc
        for gi, si, c in ((g0i, s0i, 0), (g1i, s1i, 1)):
            pltpu.sync_copy(inds_hbm.at[pl.ds(npairs + t * ppt + c * hpt, hpt)],
                            gi.at[pl.ds(0, hpt)])
            pltpu.sync_copy(inds_hbm.at[pl.ds(t * ppt + c * hpt, hpt)],
                            gi.at[pl.ds(hpt, hpt)])
            pltpu.sync_copy(inds_hbm.at[pl.ds(t * ppt + c * hpt, hpt)],
                            si.at[pl.ds(0, hpt)])
            pltpu.sync_copy(inds_hbm.at[pl.ds(npairs + t * ppt + c * hpt, hpt)],
                            si.at[pl.ds(hpt, hpt)])

        sl = pl.ds(t * slw, slw)

        def stage(k, row, sem):
            pltpu.async_copy(img_hbm.at[ch0 + k, sl], row.at[sl], sem)

        def stage_wait(k, row, sem):
            pltpu.make_async_copy(img_hbm.at[ch0 + k, sl], row.at[sl],
                                  sem).wait()

        def wb_drain(k, row, sem):
            pltpu.make_async_copy(row.at[sl], out_hbm.at[ch0 + k, sl],
                                  sem).wait()

        def gathers(row):
            pltpu.async_copy(row.at[g0i], v0, sg0)
            pltpu.async_copy(row.at[g1i], v1, sg1)

        def phase(k, X, soutX, Y, sinY, Z, sinZ, soutZ):
            pltpu.make_async_copy(X.at[g0i], v0, sg0).wait()
            pltpu.async_copy(v0, X.at[s0i], ss)
            pltpu.make_async_copy(X.at[g1i], v1, sg1).wait()
            pltpu.async_copy(v1, X.at[s1i], ss)

            @pl.when(k >= 1)
            def _():
                wb_drain(k - 1, Z, soutZ)

            @pl.when(k + 2 < cpc)
            def _():
                stage(k + 2, Z, sinZ)

            pltpu.make_async_copy(v0, X.at[s0i], ss).wait()
            pltpu.make_async_copy(v1, X.at[s1i], ss).wait()

            @pl.when(k + 1 < cpc)
            def _():
                stage_wait(k + 1, Y, sinY)

            plsc.subcore_barrier()
            pltpu.async_copy(X.at[sl], out_hbm.at[ch0 + k, sl], soutX)

            @pl.when(k + 1 < cpc)
            def _():
                gathers(Y)

        stage(0, rowA, sinA)
        stage(1, rowB, sinB)
        stage_wait(0, rowA, sinA)
        plsc.subcore_barrier()
        gathers(rowA)

        def body(k3, carry):
            k = 3 * k3
            phase(k, rowA, soutA, rowB, sinB, rowC, sinC, soutC)
            phase(k + 1, rowB, soutB, rowC, sinC, rowA, sinA, soutA)
            phase(k + 2, rowC, soutC, rowA, sinA, rowB, sinB, soutB)
            return carry

        lax.fori_loop(0, cpc // 3, body, 0)
        wb_drain(cpc - 1, rowC, soutC)

    return run(img2, inds2)


def kernel(img, inds):
    C, H, W = img.shape
    HW = H * W
    npairs = inds.shape[0] // 2
    r, c = inds // W, inds % W
    phys = (((r >> 3) * (W // 128) + (c >> 7)) << 10) | ((r & 7) << 7) | (c & 127)
    flat = (img.reshape(C, H // 8, 8, W // 128, 128)
            .transpose(0, 1, 3, 2, 4)
            .reshape(C, HW))
    out = _shuffle(flat, phys, C=C, HW=HW, npairs=npairs)
    return (out.reshape(C, H // 8, W // 128, 8, 128)
            .transpose(0, 1, 3, 2, 4)
            .reshape(C, H, W))
